# Initial kernel scaffold; baseline (speedup 1.0000x reference)
import jax
import jax.numpy as jnp
from jax.experimental import pallas as pl


def kernel(embed):
    raise NotImplementedError("write your pallas kernel here")


# speedup vs baseline: 9.5214x; 9.5214x over previous
import jax
import jax.numpy as jnp
from jax.experimental import pallas as pl
from jax.experimental.pallas import tpu as pltpu

_H = 512
_W = 512
_BAND = 16
_K = 32
_R = 16.0


def _raster_kernel(starts_ref, ncks_ref, params_ref, out_ref):
    b = pl.program_id(0)
    start = starts_ref[b]
    nck = ncks_ref[b]
    xs = jax.lax.broadcasted_iota(jnp.int32, (1, _W), 1).astype(jnp.float32) + 0.5
    y0 = (b * _BAND).astype(jnp.float32)
    out_ref[...] = jnp.zeros_like(out_ref)

    def body(i, carry):
        off = start + i * _K
        p = params_ref[pl.ds(off, _K), :]
        cx = p[:, 0:1]
        cy = p[:, 1:2]
        c0 = p[:, 2:3]
        c1 = p[:, 3:4]
        c2 = p[:, 4:5]
        w = p[:, 5:8]
        dx = xs - cx
        a = (0.5 * c0) * dx * dx
        c1dx = c1 * dx
        for y in range(_BAND):
            dy = (y0 + (y + 0.5)) - cy
            sig = a + (0.5 * c2) * (dy * dy) + dy * c1dx
            alpha = jnp.exp(-sig)
            contrib = jax.lax.dot_general(
                w, alpha, (((0,), (0,)), ((), ())),
                preferred_element_type=jnp.float32)
            out_ref[y, :, :] += contrib
        return carry

    jax.lax.fori_loop(0, nck, body, 0)


def kernel(embed):
    e = embed.reshape(-1, 9).astype(jnp.float32)
    n = e.shape[0]
    xy = jnp.tanh(e[:, :2])
    cx = 0.5 * _W * (xy[:, 0] + 1.0)
    cy = 0.5 * _H * (xy[:, 1] + 1.0)
    l0 = e[:, 5] + 0.5
    l1 = e[:, 6]
    l2 = e[:, 7] + 0.5
    cov00 = l0 * l0
    cov01 = l0 * l1
    cov11 = l1 * l1 + l2 * l2
    det = cov00 * cov11 - cov01 * cov01
    conic0 = cov11 / det
    conic1 = -cov01 / det
    conic2 = cov00 / det
    w = e[:, 2:5] * jax.nn.sigmoid(e[:, 8:9])

    order = jnp.argsort(cy)
    P = jnp.concatenate(
        [jnp.stack([cx, cy, conic0, conic1, conic2], axis=1), w], axis=1)
    P = P[order]
    cys = cy[order]

    npad = ((n + _K - 1) // _K) * _K
    if npad != n:
        P = jnp.concatenate(
            [P, jnp.zeros((npad - n, 8), dtype=P.dtype)], axis=0)

    nbands = _H // _BAND
    ylo = _BAND * jnp.arange(nbands, dtype=jnp.float32) + 0.5 - _R
    yhi = ylo + (_BAND - 1) + 2.0 * _R
    starts = jnp.searchsorted(cys, ylo).astype(jnp.int32)
    ends = jnp.searchsorted(cys, yhi, side='right').astype(jnp.int32)
    starts_al = (starts // _K) * _K
    ncks = (ends - starts_al + _K - 1) // _K

    grid_spec = pltpu.PrefetchScalarGridSpec(
        num_scalar_prefetch=2,
        grid=(nbands,),
        in_specs=[pl.BlockSpec((npad, 8), lambda b, *_: (0, 0))],
        out_specs=pl.BlockSpec((_BAND, 3, _W), lambda b, *_: (b, 0, 0)),
    )
    out = pl.pallas_call(
        _raster_kernel,
        grid_spec=grid_spec,
        out_shape=jax.ShapeDtypeStruct((_H, 3, _W), jnp.float32),
    )(starts_al, ncks, P)
    return jnp.transpose(out, (1, 0, 2))[None]
